# baseline (device time: 16214 ns/iter reference)
import jax
import jax.numpy as jnp
from jax import lax
from jax.experimental import pallas as pl
from jax.experimental.pallas import tpu as pltpu

B, SQ, SKV, D = 2, 128, 128, 512
HQ_LOC, DH = 8, 64


def kernel(x, Wq, Wo, K_ext, V_ext):
    me = lax.axis_index("i")
    Kc = lax.dynamic_slice(K_ext.reshape(B, SKV, 32 * DH),
                           (0, 0, me * HQ_LOC * DH), (B, SKV, HQ_LOC * DH))
    Vc = lax.dynamic_slice(V_ext.reshape(B, SKV, 32 * DH),
                           (0, 0, me * HQ_LOC * DH), (B, SKV, HQ_LOC * DH))
    x = x.astype(jnp.bfloat16)
    Wq = Wq.astype(jnp.bfloat16)
    Wo = Wo.astype(jnp.bfloat16)
    Kc = Kc.astype(jnp.bfloat16)
    Vc = Vc.astype(jnp.bfloat16)

    def body(x_ref, wq_ref, wo_ref, k_ref, v_ref, out_ref,
             attn_ref, acc_ref, send_ref, recv_ref, send_sems, recv_sems):
        my_pos = lax.axis_index("i")
        partners = [my_pos ^ 1, 3 - my_pos]

        barrier_sem = pltpu.get_barrier_semaphore()
        for p in partners:
            pl.semaphore_signal(barrier_sem, 1, device_id=(p,),
                                device_id_type=pl.DeviceIdType.MESH)

        def rows(c):
            return pl.ds(c * SQ, SQ)

        def compute_chunk(c):
            q2 = jnp.dot(x_ref[c], wq_ref[:],
                         preferred_element_type=jnp.float32
                         ).astype(jnp.bfloat16)
            kc = k_ref[c]
            vc = v_ref[c]
            for h in range(HQ_LOC):
                q = q2[:, h * DH:(h + 1) * DH]
                k = kc[:, h * DH:(h + 1) * DH]
                v = vc[:, h * DH:(h + 1) * DH]
                s = lax.dot_general(
                    q, k, (((1,), (1,)), ((), ())),
                    preferred_element_type=jnp.float32,
                ) * 0.125
                p = jnp.exp(s)
                l = jnp.sum(p, axis=1, keepdims=True)
                o = jnp.dot(p.astype(jnp.bfloat16), v,
                            preferred_element_type=jnp.float32) / l
                attn_ref[rows(c), h * DH:(h + 1) * DH] = o.astype(jnp.bfloat16)
            acc_ref[rows(c), :] = jnp.dot(
                attn_ref[rows(c), :], wo_ref[:],
                preferred_element_type=jnp.float32)

        def start_rdma(c, step):
            send_ref[step, c] = acc_ref[rows(c), :].astype(jnp.bfloat16)
            rdma = pltpu.make_async_remote_copy(
                src_ref=send_ref.at[step, c],
                dst_ref=recv_ref.at[step, c],
                send_sem=send_sems.at[step, c],
                recv_sem=recv_sems.at[step, c],
                device_id=(partners[step ^ c],),
                device_id_type=pl.DeviceIdType.MESH,
            )
            rdma.start()
            return rdma

        compute_chunk(0)
        pl.semaphore_wait(barrier_sem, 2)
        r00 = start_rdma(0, 0)
        compute_chunk(1)
        r01 = start_rdma(1, 0)

        r00.wait()
        acc_ref[rows(0), :] = (acc_ref[rows(0), :]
                               + recv_ref[0, 0].astype(jnp.float32))
        r10 = start_rdma(0, 1)

        r01.wait()
        acc_ref[rows(1), :] = (acc_ref[rows(1), :]
                               + recv_ref[0, 1].astype(jnp.float32))
        r11 = start_rdma(1, 1)

        r10.wait()
        out_ref[0, :, :] = (acc_ref[rows(0), :]
                            + recv_ref[1, 0].astype(jnp.float32))
        r11.wait()
        out_ref[1, :, :] = (acc_ref[rows(1), :]
                            + recv_ref[1, 1].astype(jnp.float32))

    return pl.pallas_call(
        body,
        out_shape=jax.ShapeDtypeStruct((B, SQ, D), jnp.float32),
        in_specs=[pl.BlockSpec(memory_space=pltpu.VMEM)] * 5,
        out_specs=pl.BlockSpec(memory_space=pltpu.VMEM),
        scratch_shapes=[
            pltpu.VMEM((B * SQ, D), jnp.bfloat16),
            pltpu.VMEM((B * SQ, D), jnp.float32),
            pltpu.VMEM((2, B, SQ, D), jnp.bfloat16),
            pltpu.VMEM((2, B, SQ, D), jnp.bfloat16),
            pltpu.SemaphoreType.DMA((2, 2)),
            pltpu.SemaphoreType.DMA((2, 2)),
        ],
        compiler_params=pltpu.CompilerParams(collective_id=0),
    )(x, Wq, Wo, Kc, Vc)


# device time: 13853 ns/iter; 1.1704x vs baseline; 1.1704x over previous
import jax
import jax.numpy as jnp
from jax import lax
from jax.experimental import pallas as pl
from jax.experimental.pallas import tpu as pltpu

B, SQ, SKV, D = 2, 128, 128, 512
HQ_LOC, DH = 8, 64


def kernel(x, Wq, Wo, K_ext, V_ext):
    me = lax.axis_index("i")
    Kc = lax.dynamic_slice(K_ext.reshape(B, SKV, 32 * DH),
                           (0, 0, me * HQ_LOC * DH), (B, SKV, HQ_LOC * DH))
    Vc = lax.dynamic_slice(V_ext.reshape(B, SKV, 32 * DH),
                           (0, 0, me * HQ_LOC * DH), (B, SKV, HQ_LOC * DH))

    def body(x_ref, wq_ref, wo_ref, k_ref, v_ref, out_ref,
             attn_ref, acc_ref, send_ref, recv_ref, send_sems, recv_sems):
        my_pos = lax.axis_index("i")
        partners = [my_pos ^ 1, 3 - my_pos]

        barrier_sem = pltpu.get_barrier_semaphore()
        for p in partners:
            pl.semaphore_signal(barrier_sem, 1, device_id=(p,),
                                device_id_type=pl.DeviceIdType.MESH)

        def rows(c):
            return pl.ds(c * SQ, SQ)

        def compute_chunk(c):
            q2 = jnp.dot(x_ref[c], wq_ref[:],
                         preferred_element_type=jnp.float32)
            kc = k_ref[c]
            vc = v_ref[c]
            for h in range(HQ_LOC):
                q = q2[:, h * DH:(h + 1) * DH]
                k = kc[:, h * DH:(h + 1) * DH]
                v = vc[:, h * DH:(h + 1) * DH]
                s = lax.dot_general(
                    q, k, (((1,), (1,)), ((), ())),
                    preferred_element_type=jnp.float32,
                ) * 0.125
                p = jnp.exp(s)
                l = jnp.sum(p, axis=1, keepdims=True)
                o = jnp.dot(p, v,
                            preferred_element_type=jnp.float32) / l
                attn_ref[rows(c), h * DH:(h + 1) * DH] = o
            acc_ref[rows(c), :] = jnp.dot(
                attn_ref[rows(c), :], wo_ref[:],
                preferred_element_type=jnp.float32)

        HALF = SQ // 2

        def rows64(q):
            return pl.ds(q * HALF, HALF)

        def start_rdma(q, step):
            send_ref[step, q] = acc_ref[rows64(q), :].astype(jnp.bfloat16)
            rdma = pltpu.make_async_remote_copy(
                src_ref=send_ref.at[step, q],
                dst_ref=recv_ref.at[step, q],
                send_sem=send_sems.at[step, q],
                recv_sem=recv_sems.at[step, q],
                device_id=(partners[step ^ (q % 2)],),
                device_id_type=pl.DeviceIdType.MESH,
            )
            rdma.start()
            return rdma

        r0 = [None] * 4
        compute_chunk(0)
        pl.semaphore_wait(barrier_sem, 2)
        r0[0] = start_rdma(0, 0)
        r0[1] = start_rdma(1, 0)
        compute_chunk(1)
        r0[2] = start_rdma(2, 0)
        r0[3] = start_rdma(3, 0)

        r1 = [None] * 4
        for q in range(4):
            r0[q].wait()
            acc_ref[rows64(q), :] = (acc_ref[rows64(q), :]
                                     + recv_ref[0, q].astype(jnp.float32))
            r1[q] = start_rdma(q, 1)

        for q in range(4):
            r1[q].wait()
            out_ref[q // 2, (q % 2) * HALF:(q % 2) * HALF + HALF, :] = (
                acc_ref[rows64(q), :] + recv_ref[1, q].astype(jnp.float32))

    return pl.pallas_call(
        body,
        out_shape=jax.ShapeDtypeStruct((B, SQ, D), jnp.float32),
        in_specs=[pl.BlockSpec(memory_space=pltpu.VMEM)] * 5,
        out_specs=pl.BlockSpec(memory_space=pltpu.VMEM),
        scratch_shapes=[
            pltpu.VMEM((B * SQ, D), jnp.float32),
            pltpu.VMEM((B * SQ, D), jnp.float32),
            pltpu.VMEM((2, 4, SQ // 2, D), jnp.bfloat16),
            pltpu.VMEM((2, 4, SQ // 2, D), jnp.bfloat16),
            pltpu.SemaphoreType.DMA((2, 4)),
            pltpu.SemaphoreType.DMA((2, 4)),
        ],
        compiler_params=pltpu.CompilerParams(collective_id=0),
    )(x, Wq, Wo, Kc, Vc)
